# baseline (speedup 1.0000x reference)
import functools

import jax
import jax.numpy as jnp
from jax import lax
from jax.experimental import pallas as pl
from jax.experimental.pallas import tpu as pltpu
from jax.experimental.pallas import tpu_sc as plsc

_B, _L = 1024, 200
_D = 100
_DP = 128
_N = _B * _L
_INFO = plsc.get_sparse_core_info()
_NC, _NS = _INFO.num_cores, _INFO.num_subcores
_NW = _NC * _NS
_PER_W = _N // _NW
_G = 128

_mesh = plsc.VectorSubcoreMesh(core_axis_name="c", subcore_axis_name="s")


def _make_gather(n_tables, chunk):
    nsub = chunk // _G
    nchunk = _PER_W // chunk
    assert chunk % _G == 0 and _PER_W % chunk == 0

    out_type = tuple(
        jax.ShapeDtypeStruct((_N, _DP), jnp.float32) for _ in range(n_tables))
    scratch = [
        pltpu.VMEM((_PER_W,), jnp.int32),
        pltpu.VMEM((chunk, _DP), jnp.float32),
        pltpu.SemaphoreType.DMA,
    ]

    @functools.partial(
        pl.kernel, mesh=_mesh, out_type=out_type, scratch_types=scratch)
    def gather(*refs):
        idx_hbm = refs[:n_tables]
        tabs = refs[n_tables:2 * n_tables]
        outs = refs[2 * n_tables:3 * n_tables]
        idx_v, buf, sem = refs[3 * n_tables:]

        wid = lax.axis_index("s") * _NC + lax.axis_index("c")
        wbase = wid * _PER_W

        for t in range(n_tables):
            pltpu.sync_copy(idx_hbm[t].at[pl.ds(wbase, _PER_W)], idx_v)

            def body(k, carry, t=t):
                base = wbase + k * chunk
                copies = []
                for j in range(nsub):
                    s = pl.ds(j * _G, _G)
                    copies.append(pltpu.async_copy(
                        tabs[t].at[idx_v.at[pl.ds(k * chunk + j * _G, _G)]],
                        buf.at[s], sem))
                for c in copies:
                    c.wait()
                pltpu.sync_copy(buf, outs[t].at[pl.ds(base, chunk)])
                return carry

            lax.fori_loop(0, nchunk, body, 0)

    return gather


_gather2 = _make_gather(2, 640)


def _make_gather_db(chunk):
    nsub = chunk // _G
    nchunk = _PER_W // chunk
    assert chunk % _G == 0 and _PER_W % chunk == 0

    @functools.partial(
        pl.kernel, mesh=_mesh,
        out_type=jax.ShapeDtypeStruct((_N, _DP), jnp.float32),
        scratch_types=[
            pltpu.VMEM((_PER_W,), jnp.int32),
            pltpu.VMEM((chunk, _DP), jnp.float32),
            pltpu.VMEM((chunk, _DP), jnp.float32),
            pltpu.SemaphoreType.DMA,
            pltpu.SemaphoreType.DMA,
        ])
    def gather(idx_hbm, tab, out, idx_v, b0, b1, s0, s1):
        wid = lax.axis_index("s") * _NC + lax.axis_index("c")
        wbase = wid * _PER_W
        pltpu.sync_copy(idx_hbm.at[pl.ds(wbase, _PER_W)], idx_v)

        def fire(k, buf, sem):
            return [
                pltpu.async_copy(
                    tab.at[idx_v.at[pl.ds(k * chunk + j * _G, _G)]],
                    buf.at[pl.ds(j * _G, _G)], sem)
                for j in range(nsub)
            ]

        def drain_write(k, buf, copies):
            for c in copies:
                c.wait()
            pltpu.sync_copy(buf, out.at[pl.ds(wbase + k * chunk, chunk)])

        def body_pair(k2, carry):
            k = 2 * k2
            c0 = fire(k, b0, s0)
            c1 = fire(k + 1, b1, s1)
            drain_write(k, b0, c0)
            drain_write(k + 1, b1, c1)
            return carry

        lax.fori_loop(0, nchunk // 2, body_pair, 0)
        if nchunk % 2:
            ct = fire(nchunk - 1, b0, s0)
            drain_write(nchunk - 1, b0, ct)

    return gather


_gather1w = _make_gather_db(256)


def _pad128_mxu(table):
    eye = jnp.eye(_D, _DP, dtype=jnp.float32)
    return lax.dot_general(
        table, eye, (((1,), (0,)), ((), ())),
        precision=lax.Precision.HIGHEST,
        preferred_element_type=jnp.float32,
    )


def kernel(words, tags, lemmas, word_table, tag_table, lemma_table):
    ot, ol = _gather2(
        tags.reshape(-1), lemmas.reshape(-1),
        _pad128_mxu(tag_table), _pad128_mxu(lemma_table),
    )
    ow = _gather1w(words.reshape(-1), _pad128_mxu(word_table))
    embed = jnp.concatenate([ow[:, :_D], ot[:, :_D], ol[:, :_D]], axis=-1)
    return embed.reshape(_B, _L, 3 * _D)


# speedup vs baseline: 1.0015x; 1.0015x over previous
import functools

import jax
import jax.numpy as jnp
from jax import lax
from jax.experimental import pallas as pl
from jax.experimental.pallas import tpu as pltpu
from jax.experimental.pallas import tpu_sc as plsc

_B, _L = 1024, 200
_D = 100
_DP = 128
_N = _B * _L
_INFO = plsc.get_sparse_core_info()
_NC, _NS = _INFO.num_cores, _INFO.num_subcores
_NW = _NC * _NS
_PER_W = _N // _NW
_G = 128

_mesh = plsc.VectorSubcoreMesh(core_axis_name="c", subcore_axis_name="s")


def _make_gather(n_tables, chunk):
    nsub = chunk // _G
    nchunk = _PER_W // chunk
    assert chunk % _G == 0 and _PER_W % chunk == 0

    out_type = tuple(
        jax.ShapeDtypeStruct((_N, _DP), jnp.float32) for _ in range(n_tables))
    scratch = [
        pltpu.VMEM((_PER_W,), jnp.int32),
        pltpu.VMEM((chunk, _DP), jnp.float32),
        pltpu.SemaphoreType.DMA,
    ]

    @functools.partial(
        pl.kernel, mesh=_mesh, out_type=out_type, scratch_types=scratch)
    def gather(*refs):
        idx_hbm = refs[:n_tables]
        tabs = refs[n_tables:2 * n_tables]
        outs = refs[2 * n_tables:3 * n_tables]
        idx_v, buf, sem = refs[3 * n_tables:]

        wid = lax.axis_index("s") * _NC + lax.axis_index("c")
        wbase = wid * _PER_W

        for t in range(n_tables):
            pltpu.sync_copy(idx_hbm[t].at[pl.ds(wbase, _PER_W)], idx_v)

            def body(k, carry, t=t):
                base = wbase + k * chunk
                copies = []
                for j in range(nsub):
                    s = pl.ds(j * _G, _G)
                    copies.append(pltpu.async_copy(
                        tabs[t].at[idx_v.at[pl.ds(k * chunk + j * _G, _G)]],
                        buf.at[s], sem))
                for c in copies:
                    c.wait()
                pltpu.sync_copy(buf, outs[t].at[pl.ds(base, chunk)])
                return carry

            lax.fori_loop(0, nchunk, body, 0)

    return gather


_gather2 = _make_gather(2, 640)
_gather1w = _make_gather(1, 640)


def _pad128_mxu(table):
    eye = jnp.eye(_D, _DP, dtype=jnp.float32)
    return lax.dot_general(
        table, eye, (((1,), (0,)), ((), ())),
        precision=lax.Precision.HIGHEST,
        preferred_element_type=jnp.float32,
    )


def kernel(words, tags, lemmas, word_table, tag_table, lemma_table):
    ot, ol = _gather2(
        tags.reshape(-1), lemmas.reshape(-1),
        _pad128_mxu(tag_table), _pad128_mxu(lemma_table),
    )
    ow, = _gather1w(words.reshape(-1), _pad128_mxu(word_table))
    embed = jnp.concatenate([ow[:, :_D], ot[:, :_D], ol[:, :_D]], axis=-1)
    return embed.reshape(_B, _L, 3 * _D)
